# baseline (speedup 1.0000x reference)
import functools

import jax
import jax.numpy as jnp
from jax import lax
from jax.experimental import pallas as pl
from jax.experimental.pallas import tpu as pltpu
from jax.experimental.pallas import tpu_sc as plsc

D = 64
L = 16
NC = 2
NS = 16
NW = NC * NS


def _rsqrt_vec(x):
    i = lax.bitcast_convert_type(x, jnp.int32)
    i = jnp.int32(0x5F3759DF) - (i >> 1)
    y = lax.bitcast_convert_type(i, jnp.float32)
    half = x * jnp.float32(0.5)
    for _ in range(3):
        y = y * (jnp.float32(1.5) - half * y * y)
    return y


def _make_kernel(batch):
    assert batch % (8 * NW) == 0
    b_per_w = batch // NW
    o_per_w = b_per_w // 2
    n_groups = b_per_w // L
    mesh = plsc.VectorSubcoreMesh(
        core_axis_name="c", subcore_axis_name="s",
        num_cores=NC, num_subcores=NS,
    )

    @functools.partial(
        pl.kernel,
        out_type=jax.ShapeDtypeStruct((batch // 2, 2 * D), jnp.float32),
        mesh=mesh,
        scratch_types=[
            pltpu.VMEM((b_per_w,), jnp.int32),
            pltpu.VMEM((b_per_w,), jnp.int32),
            pltpu.VMEM((b_per_w, 2 * D), jnp.float32),
            pltpu.VMEM((o_per_w, 2 * D), jnp.float32),
            pltpu.SemaphoreType.DMA,
        ],
        compiler_params=pltpu.CompilerParams(needs_layout_passes=False),
    )
    def body(x_hbm, table_hbm, out_hbm, idx_v, par_v, rows_v, out_v, sem):
        wid = lax.axis_index("s") * NC + lax.axis_index("c")
        base = wid * b_per_w
        pltpu.sync_copy(x_hbm.at[pl.ds(base, b_per_w)], idx_v)

        def split_fn(k, carry):
            v = idx_v[pl.ds(k * L, L)]
            par_v[pl.ds(k * L, L)] = (v & 1) << 6
            idx_v[pl.ds(k * L, L)] = v >> 1
            return carry
        lax.fori_loop(0, n_groups, split_fn, 0)

        pltpu.async_copy(table_hbm.at[idx_v], rows_v, sem).wait()

        iot = lax.iota(jnp.int32, L)

        def group_fn(g, carry):
            lr = g * L + iot
            coff = par_v[pl.ds(g * L, L)]
            orow = lr >> 1
            ocol = (lr & 1) << 6
            acc = jnp.zeros((L,), jnp.float32)
            vs_cols = []
            for j in range(D):
                v = plsc.load_gather(rows_v, [lr, coff + j])
                acc = acc + v * v
            rs = jnp.minimum(_rsqrt_vec(acc), jnp.float32(1e12))
            for j in range(D):
                v = plsc.load_gather(rows_v, [lr, coff + j])
                plsc.store_scatter(out_v, [orow, ocol + j], v * rs)
            return carry

        lax.fori_loop(0, n_groups, group_fn, 0)
        pltpu.sync_copy(out_v, out_hbm.at[pl.ds(wid * o_per_w, o_per_w)])

    return body


def kernel(X, table):
    batch = X.shape[0]
    num_emb, d = table.shape
    table2 = table.reshape(num_emb // 2, 2 * d)
    out2 = _make_kernel(batch)(X.astype(jnp.int32), table2)
    return out2.reshape(batch, d)


# speedup vs baseline: 3.6649x; 3.6649x over previous
import functools

import jax
import jax.numpy as jnp
from jax import lax
from jax.experimental import pallas as pl
from jax.experimental.pallas import tpu as pltpu
from jax.experimental.pallas import tpu_sc as plsc

D = 64
L = 16
NC = 2
NS = 16
NW = NC * NS
BLKW = 256
OUTW = 128
SPARE = 8

B_BITS = 14
IC_SHIFT = B_BITS
BL_SHIFT = B_BITS + 8


def _rsqrt_vec(x):
    i = lax.bitcast_convert_type(x, jnp.int32)
    i = jnp.int32(0x5F3759DF) - (i >> 1)
    y = lax.bitcast_convert_type(i, jnp.float32)
    half = x * jnp.float32(0.5)
    for _ in range(3):
        y = y * (jnp.float32(1.5) - half * y * y)
    return y


def _splat(s):
    return jnp.broadcast_to(s, (L,))


def _perm(v, idx):
    return jnp.take_along_axis(v, idx, axis=0, mode="promise_in_bounds")


def _make_kernel(batch, num_emb):
    nblk = num_emb // BLKW
    tail_start = nblk * BLKW
    tail_off = tail_start - (num_emb - OUTW)
    n_vecs = batch // L
    cap = ((nblk // NW + 2 + L - 1) // L) * L
    out_rows = batch + SPARE
    mesh = plsc.VectorSubcoreMesh(
        core_axis_name="c", subcore_axis_name="s",
        num_cores=NC, num_subcores=NS,
    )

    iota = lambda: lax.iota(jnp.int32, L)

    @functools.partial(
        pl.kernel,
        out_type=jax.ShapeDtypeStruct((out_rows, OUTW), jnp.float32),
        mesh=mesh,
        scratch_types=[
            pltpu.VMEM((batch,), jnp.int32),
            pltpu.VMEM((batch,), jnp.int32),
            pltpu.VMEM((batch,), jnp.int32),
            pltpu.VMEM((cap,), jnp.int32),
            pltpu.VMEM((cap,), jnp.int32),
            pltpu.VMEM((cap,), jnp.int32),
            pltpu.VMEM((D, OUTW), jnp.float32),
            pltpu.VMEM((D, BLKW), jnp.float32),
            pltpu.VMEM((D, BLKW), jnp.float32),
            pltpu.VMEM((D, OUTW), jnp.float32),
            pltpu.VMEM((D,), jnp.int32),
            pltpu.SemaphoreType.DMA,
            pltpu.SemaphoreType.DMA,
        ],
        compiler_params=pltpu.CompilerParams(needs_layout_passes=False),
    )
    def body(x_hbm, tablet_hbm, tail_hbm, out_hbm,
             xv, wl, wl2, counts, offs_a, offs_b,
             tail_v, blk0, blk1, rowbuf, pos_ring, sem0, sem1):
        wid = lax.axis_index("s") * NC + lax.axis_index("c")
        c_lo = (wid * nblk) // NW
        c_hi = ((wid + 1) * nblk) // NW
        n_local = c_hi - c_lo
        is_last = wid == NW - 1
        spare = batch + (wid & (SPARE - 1))

        pltpu.sync_copy(x_hbm, xv)
        pltpu.sync_copy(tail_hbm, tail_v)

        def start_blk(bl, buf, sem):
            coff = pl.multiple_of((c_lo + bl) * BLKW, OUTW)
            pltpu.make_async_copy(
                tablet_hbm.at[:, pl.ds(coff, BLKW)], buf, sem
            ).start()

        def wait_blk(bl, buf, sem):
            coff = pl.multiple_of((c_lo + bl) * BLKW, OUTW)
            pltpu.make_async_copy(
                tablet_hbm.at[:, pl.ds(coff, BLKW)], buf, sem
            ).wait()

        start_blk(0, blk0, sem0)
        start_blk(1, blk1, sem1)

        lo_v = _splat(c_lo)
        hi_v = _splat(c_hi)
        last_v = jnp.broadcast_to(is_last, (L,))
        nblk_v = _splat(jnp.int32(nblk))

        def scan_fn(o, off):
            v = xv[pl.ds(o * L, L)]
            tc = v >> 8
            m = ((tc >= lo_v) & (tc < hi_v)) | (last_v & (tc == nblk_v))
            word = ((iota() + o * L)
                    | ((v & 255) << IC_SHIFT)
                    | ((tc - lo_v) << BL_SHIFT))
            plsc.store_compressed(wl.at[pl.ds(off[0], L)], word, mask=m)
            return off + plsc.all_reduce_population_count(m)

        offv = lax.fori_loop(0, n_vecs, scan_fn, _splat(jnp.int32(0)))
        count = offv[0]

        zero_v = _splat(jnp.int32(0))
        for m in range(cap // L):
            counts[pl.ds(m * L, L)] = zero_v

        lane0 = iota() == 0

        def _entry(ref, e):
            base = (e >> 4) << 4
            v = ref[pl.ds(base, L)]
            return _perm(v, _splat(e & 15))

        def count_fn(e, carry):
            blv = _entry(wl, e) >> BL_SHIFT
            c = plsc.load_gather(counts, [blv])
            plsc.store_scatter(counts, [blv], c + 1, mask=lane0)
            return carry

        lax.fori_loop(0, count, count_fn, 0)

        carry = zero_v
        for m in range(cap // L):
            c = counts[pl.ds(m * L, L)]
            cs = plsc.cumsum(c)
            excl = cs - c + carry
            offs_a[pl.ds(m * L, L)] = excl
            offs_b[pl.ds(m * L, L)] = excl
            carry = carry + _splat(cs[L - 1])

        def place_fn(e, carry):
            w = _entry(wl, e)
            blv = w >> BL_SHIFT
            ov = plsc.load_gather(offs_b, [blv])
            plsc.store_scatter(wl2, [ov], w, mask=lane0)
            plsc.store_scatter(offs_b, [blv], ov + 1, mask=lane0)
            return carry

        lax.fori_loop(0, count, place_fn, 0)

        jvecs = [iota() + m * L for m in range(D // L)]
        spare_v = _splat(spare)

        def flush():
            pltpu.sync_copy(rowbuf, out_hbm.at[pos_ring])

        def process(start_e, end_e, buf, k0, col_off=0):
            def ebody(e, k):
                w = _entry(wl2, e)
                bv = w & ((1 << B_BITS) - 1)
                ic = ((w >> IC_SHIFT) & 255) + col_off
                vs = [plsc.load_gather(buf, [jv, ic]) for jv in jvecs]
                ss = vs[0] * vs[0]
                for v in vs[1:]:
                    ss = ss + v * v
                for msk in (8, 4, 2, 1):
                    ss = ss + _perm(ss, iota() ^ msk)
                rs = jnp.minimum(_rsqrt_vec(ss), jnp.float32(1e12))
                s = k & 63
                sv = _splat(s)
                for jv, v in zip(jvecs, vs):
                    plsc.store_scatter(rowbuf, [sv, jv], v * rs)
                plsc.store_scatter(pos_ring, [sv], bv, mask=lane0)

                @pl.when(s == 63)
                def _():
                    flush()

                return k + 1

            return lax.fori_loop(start_e, end_e, ebody, k0)

        def bucket_bounds(bl):
            st = plsc.load_gather(offs_a, [_splat(bl)])[0]
            cn = plsc.load_gather(counts, [_splat(bl)])[0]
            return st, st + cn

        def pair_fn(h, k):
            b0 = 2 * h
            b1 = 2 * h + 1
            wait_blk(b0, blk0, sem0)
            st0, en0 = bucket_bounds(b0)
            k = process(st0, en0, blk0, k)

            @pl.when(b1 + 1 < n_local)
            def _():
                start_blk(b1 + 1, blk0, sem0)

            @pl.when(b1 < n_local)
            def _():
                wait_blk(b1, blk1, sem1)

            st1, en1 = bucket_bounds(b1)
            en1 = jnp.where(b1 < n_local, en1, st1)
            k = process(st1, en1, blk1, k)

            @pl.when(b1 + 2 < n_local)
            def _():
                start_blk(b1 + 2, blk1, sem1)

            return k

        k = lax.fori_loop(0, (n_local + 1) >> 1, pair_fn, jnp.int32(0))

        st_t = plsc.load_gather(offs_a, [_splat(n_local)])[0]
        k = process(st_t, count, tail_v, k, col_off=tail_off)

        rem = k & 63

        @pl.when(rem > 0)
        def _():
            rv = _splat(rem)
            for m in range(D // L):
                pv = pos_ring[pl.ds(m * L, L)]
                lanes = iota() + m * L
                pos_ring[pl.ds(m * L, L)] = jnp.where(
                    lanes < rv, pv, spare_v)
            flush()

    return body


def kernel(X, table):
    batch = X.shape[0]
    num_emb, d = table.shape
    tablet = table.T
    tail = lax.slice(table, (num_emb - OUTW, 0), (num_emb, d)).T
    out = _make_kernel(batch, num_emb)(X.astype(jnp.int32), tablet, tail)
    return out[:batch, :d]


# speedup vs baseline: 4.1686x; 1.1374x over previous
import functools

import jax
import jax.numpy as jnp
from jax import lax
from jax.experimental import pallas as pl
from jax.experimental.pallas import tpu as pltpu
from jax.experimental.pallas import tpu_sc as plsc

D = 64
L = 16
NC = 2
NS = 16
NW = NC * NS
BLKW = 256
OUTW = 128
SPARE = 8

B_BITS = 14
IC_SHIFT = B_BITS
BL_SHIFT = B_BITS + 8


def _rsqrt_vec(x):
    i = lax.bitcast_convert_type(x, jnp.int32)
    i = jnp.int32(0x5F3759DF) - (i >> 1)
    y = lax.bitcast_convert_type(i, jnp.float32)
    half = x * jnp.float32(0.5)
    for _ in range(3):
        y = y * (jnp.float32(1.5) - half * y * y)
    return y


def _splat(s):
    return jnp.broadcast_to(s, (L,))


def _perm(v, idx):
    return jnp.take_along_axis(v, idx, axis=0, mode="promise_in_bounds")


def _make_kernel(batch, num_emb):
    nblk = num_emb // BLKW
    tail_start = nblk * BLKW
    tail_off = tail_start - (num_emb - OUTW)
    n_vecs = batch // L
    cap = ((nblk // NW + 2 + L - 1) // L) * L
    out_rows = batch + SPARE
    mesh = plsc.VectorSubcoreMesh(
        core_axis_name="c", subcore_axis_name="s",
        num_cores=NC, num_subcores=NS,
    )

    iota = lambda: lax.iota(jnp.int32, L)

    @functools.partial(
        pl.kernel,
        out_type=jax.ShapeDtypeStruct((out_rows, OUTW), jnp.float32),
        mesh=mesh,
        scratch_types=[
            pltpu.VMEM((batch,), jnp.int32),
            pltpu.VMEM((batch,), jnp.int32),
            pltpu.VMEM((batch,), jnp.int32),
            pltpu.VMEM((cap,), jnp.int32),
            pltpu.VMEM((cap,), jnp.int32),
            pltpu.VMEM((cap,), jnp.int32),
            pltpu.VMEM((D, OUTW), jnp.float32),
            pltpu.VMEM((D, BLKW), jnp.float32),
            pltpu.VMEM((D, BLKW), jnp.float32),
            pltpu.VMEM((D, BLKW), jnp.float32),
            pltpu.VMEM((D, OUTW), jnp.float32),
            pltpu.VMEM((D,), jnp.int32),
            pltpu.SemaphoreType.DMA,
            pltpu.SemaphoreType.DMA,
            pltpu.SemaphoreType.DMA,
        ],
        compiler_params=pltpu.CompilerParams(needs_layout_passes=False),
    )
    def body(x_hbm, tablet_hbm, tail_hbm, out_hbm,
             xv, wl, wl2, counts, offs_a, offs_b,
             tail_v, blk0, blk1, blk2, rowbuf, pos_ring,
             sem0, sem1, sem2):
        wid = lax.axis_index("s") * NC + lax.axis_index("c")
        c_lo = (wid * nblk) // NW
        c_hi = ((wid + 1) * nblk) // NW
        n_local = c_hi - c_lo
        is_last = wid == NW - 1
        spare = batch + (wid & (SPARE - 1))

        pltpu.sync_copy(x_hbm, xv)
        pltpu.sync_copy(tail_hbm, tail_v)

        def start_blk(bl, buf, sem):
            coff = pl.multiple_of((c_lo + bl) * BLKW, OUTW)
            pltpu.make_async_copy(
                tablet_hbm.at[:, pl.ds(coff, BLKW)], buf, sem
            ).start()

        def wait_blk(bl, buf, sem):
            coff = pl.multiple_of((c_lo + bl) * BLKW, OUTW)
            pltpu.make_async_copy(
                tablet_hbm.at[:, pl.ds(coff, BLKW)], buf, sem
            ).wait()

        bufs = (blk0, blk1, blk2)
        sems = (sem0, sem1, sem2)
        NBUF = 3
        for q in range(NBUF):
            start_blk(q, bufs[q], sems[q])

        lo_v = _splat(c_lo)
        hi_v = _splat(c_hi)
        last_v = jnp.broadcast_to(is_last, (L,))
        nblk_v = _splat(jnp.int32(nblk))

        def scan_fn(o, off):
            v = xv[pl.ds(o * L, L)]
            tc = v >> 8
            m = ((tc >= lo_v) & (tc < hi_v)) | (last_v & (tc == nblk_v))
            word = ((iota() + o * L)
                    | ((v & 255) << IC_SHIFT)
                    | ((tc - lo_v) << BL_SHIFT))
            plsc.store_compressed(wl.at[pl.ds(off[0], L)], word, mask=m)
            return off + plsc.all_reduce_population_count(m)

        offv = lax.fori_loop(0, n_vecs, scan_fn, _splat(jnp.int32(0)))
        count = offv[0]

        zero_v = _splat(jnp.int32(0))
        for m in range(cap // L):
            counts[pl.ds(m * L, L)] = zero_v

        lane0 = iota() == 0

        def _entry(ref, e):
            base = (e >> 4) << 4
            v = ref[pl.ds(base, L)]
            return _perm(v, _splat(e & 15))

        def count_fn(e, carry):
            blv = _entry(wl, e) >> BL_SHIFT
            c = plsc.load_gather(counts, [blv])
            plsc.store_scatter(counts, [blv], c + 1, mask=lane0)
            return carry

        lax.fori_loop(0, count, count_fn, 0)

        carry = zero_v
        for m in range(cap // L):
            c = counts[pl.ds(m * L, L)]
            cs = plsc.cumsum(c)
            excl = cs - c + carry
            offs_a[pl.ds(m * L, L)] = excl
            offs_b[pl.ds(m * L, L)] = excl
            carry = carry + _splat(cs[L - 1])

        def place_fn(e, carry):
            w = _entry(wl, e)
            blv = w >> BL_SHIFT
            ov = plsc.load_gather(offs_b, [blv])
            plsc.store_scatter(wl2, [ov], w, mask=lane0)
            plsc.store_scatter(offs_b, [blv], ov + 1, mask=lane0)
            return carry

        lax.fori_loop(0, count, place_fn, 0)

        jvecs = [iota() + m * L for m in range(D // L)]
        spare_v = _splat(spare)

        def flush():
            pltpu.sync_copy(rowbuf, out_hbm.at[pos_ring])

        def process(start_e, end_e, buf, k0, col_off=0):
            def ebody(e, k):
                w = _entry(wl2, e)
                bv = w & ((1 << B_BITS) - 1)
                ic = ((w >> IC_SHIFT) & 255) + col_off
                vs = [plsc.load_gather(buf, [jv, ic]) for jv in jvecs]
                ss = vs[0] * vs[0]
                for v in vs[1:]:
                    ss = ss + v * v
                for msk in (8, 4, 2, 1):
                    ss = ss + _perm(ss, iota() ^ msk)
                rs = jnp.minimum(_rsqrt_vec(ss), jnp.float32(1e12))
                s = k & 63
                sv = _splat(s)
                for jv, v in zip(jvecs, vs):
                    plsc.store_scatter(rowbuf, [sv, jv], v * rs)
                plsc.store_scatter(pos_ring, [sv], bv, mask=lane0)

                @pl.when(s == 63)
                def _():
                    flush()

                return k + 1

            return lax.fori_loop(start_e, end_e, ebody, k0)

        def bucket_bounds(bl):
            st = plsc.load_gather(offs_a, [_splat(bl)])[0]
            cn = plsc.load_gather(counts, [_splat(bl)])[0]
            return st, st + cn

        def quad_fn(h, k):
            for q in range(NBUF):
                bq = NBUF * h + q

                @pl.when(bq < n_local)
                def _():
                    wait_blk(bq, bufs[q], sems[q])

                stq, enq = bucket_bounds(bq)
                enq = jnp.where(bq < n_local, enq, stq)
                k = process(stq, enq, bufs[q], k)

                @pl.when(bq + NBUF < n_local)
                def _():
                    start_blk(bq + NBUF, bufs[q], sems[q])

            return k

        k = lax.fori_loop(
            0, (n_local + NBUF - 1) // NBUF, quad_fn, jnp.int32(0))

        st_t = plsc.load_gather(offs_a, [_splat(n_local)])[0]
        k = process(st_t, count, tail_v, k, col_off=tail_off)

        rem = k & 63

        @pl.when(rem > 0)
        def _():
            rv = _splat(rem)
            for m in range(D // L):
                pv = pos_ring[pl.ds(m * L, L)]
                lanes = iota() + m * L
                pos_ring[pl.ds(m * L, L)] = jnp.where(
                    lanes < rv, pv, spare_v)
            flush()

    return body


def kernel(X, table):
    batch = X.shape[0]
    num_emb, d = table.shape
    tablet = table.T
    tail = lax.slice(table, (num_emb - OUTW, 0), (num_emb, d)).T
    out = _make_kernel(batch, num_emb)(X.astype(jnp.int32), tablet, tail)
    return out[:batch, :d]


# speedup vs baseline: 4.2714x; 1.0247x over previous
import functools

import jax
import jax.numpy as jnp
from jax import lax
from jax.experimental import pallas as pl
from jax.experimental.pallas import tpu as pltpu
from jax.experimental.pallas import tpu_sc as plsc

D = 64
L = 16
NC = 2
NS = 16
NW = NC * NS
BLKW = 256
OUTW = 128
SPARE = 8

B_BITS = 14
IC_SHIFT = B_BITS
BL_SHIFT = B_BITS + 8


def _rsqrt_vec(x):
    i = lax.bitcast_convert_type(x, jnp.int32)
    i = jnp.int32(0x5F3759DF) - (i >> 1)
    y = lax.bitcast_convert_type(i, jnp.float32)
    half = x * jnp.float32(0.5)
    for _ in range(3):
        y = y * (jnp.float32(1.5) - half * y * y)
    return y


def _splat(s):
    return jnp.broadcast_to(s, (L,))


def _perm(v, idx):
    return jnp.take_along_axis(v, idx, axis=0, mode="promise_in_bounds")


def _make_kernel(batch, num_emb):
    nblk = num_emb // BLKW
    tail_start = nblk * BLKW
    tail_off = tail_start - (num_emb - OUTW)
    n_vecs = batch // L
    cap = ((nblk // NW + 2 + L - 1) // L) * L
    out_rows = batch + SPARE
    mesh = plsc.VectorSubcoreMesh(
        core_axis_name="c", subcore_axis_name="s",
        num_cores=NC, num_subcores=NS,
    )

    iota = lambda: lax.iota(jnp.int32, L)

    @functools.partial(
        pl.kernel,
        out_type=jax.ShapeDtypeStruct((out_rows, OUTW), jnp.float32),
        mesh=mesh,
        scratch_types=[
            pltpu.VMEM((2048,), jnp.int32),
            pltpu.VMEM((batch,), jnp.int32),
            pltpu.VMEM((batch,), jnp.int32),
            pltpu.VMEM((cap,), jnp.int32),
            pltpu.VMEM((cap,), jnp.int32),
            pltpu.VMEM((cap,), jnp.int32),
            pltpu.VMEM((D, OUTW), jnp.float32),
            pltpu.VMEM((D, BLKW), jnp.float32),
            pltpu.VMEM((D, BLKW), jnp.float32),
            pltpu.VMEM((D, BLKW), jnp.float32),
            pltpu.VMEM((D, BLKW), jnp.float32),
            pltpu.VMEM((D, OUTW), jnp.float32),
            pltpu.VMEM((D,), jnp.int32),
            pltpu.SemaphoreType.DMA,
            pltpu.SemaphoreType.DMA,
            pltpu.SemaphoreType.DMA,
            pltpu.SemaphoreType.DMA,
        ],
        compiler_params=pltpu.CompilerParams(needs_layout_passes=False),
    )
    def body(x_hbm, tablet_hbm, tail_hbm, out_hbm,
             xv, wl, wl2, counts, offs_a, offs_b,
             tail_v, blk0, blk1, blk2, blk3, rowbuf, pos_ring,
             sem0, sem1, sem2, sem3):
        wid = lax.axis_index("s") * NC + lax.axis_index("c")
        c_lo = (wid * nblk) // NW
        c_hi = ((wid + 1) * nblk) // NW
        n_local = c_hi - c_lo
        is_last = wid == NW - 1
        spare = batch + (wid & (SPARE - 1))

        pltpu.sync_copy(tail_hbm, tail_v)

        def start_blk(bl, buf, sem):
            coff = pl.multiple_of((c_lo + bl) * BLKW, OUTW)
            pltpu.make_async_copy(
                tablet_hbm.at[:, pl.ds(coff, BLKW)], buf, sem
            ).start()

        def wait_blk(bl, buf, sem):
            coff = pl.multiple_of((c_lo + bl) * BLKW, OUTW)
            pltpu.make_async_copy(
                tablet_hbm.at[:, pl.ds(coff, BLKW)], buf, sem
            ).wait()

        bufs = (blk0, blk1, blk2, blk3)
        sems = (sem0, sem1, sem2, sem3)
        NBUF = 4
        for q in range(NBUF):
            start_blk(q, bufs[q], sems[q])

        lo_v = _splat(c_lo)
        hi_v = _splat(c_hi)
        last_v = jnp.broadcast_to(is_last, (L,))
        nblk_v = _splat(jnp.int32(nblk))

        XCH = 2048

        def chunk_fn(ch, off):
            pltpu.sync_copy(x_hbm.at[pl.ds(ch * XCH, XCH)], xv)

            def scan_fn(o, off):
                v = xv[pl.ds(o * L, L)]
                tc = v >> 8
                m = ((tc >= lo_v) & (tc < hi_v)) | (
                    last_v & (tc == nblk_v))
                word = ((iota() + (ch * XCH + o * L))
                        | ((v & 255) << IC_SHIFT)
                        | ((tc - lo_v) << BL_SHIFT))
                plsc.store_compressed(
                    wl.at[pl.ds(off[0], L)], word, mask=m)
                return off + plsc.all_reduce_population_count(m)

            return lax.fori_loop(0, XCH // L, scan_fn, off)

        offv = lax.fori_loop(
            0, batch // XCH, chunk_fn, _splat(jnp.int32(0)))
        count = offv[0]

        zero_v = _splat(jnp.int32(0))
        for m in range(cap // L):
            counts[pl.ds(m * L, L)] = zero_v

        lane0 = iota() == 0

        def _entry(ref, e):
            base = (e >> 4) << 4
            v = ref[pl.ds(base, L)]
            return _perm(v, _splat(e & 15))

        def count_fn(e, carry):
            blv = _entry(wl, e) >> BL_SHIFT
            c = plsc.load_gather(counts, [blv])
            plsc.store_scatter(counts, [blv], c + 1, mask=lane0)
            return carry

        lax.fori_loop(0, count, count_fn, 0)

        carry = zero_v
        for m in range(cap // L):
            c = counts[pl.ds(m * L, L)]
            cs = plsc.cumsum(c)
            excl = cs - c + carry
            offs_a[pl.ds(m * L, L)] = excl
            offs_b[pl.ds(m * L, L)] = excl
            carry = carry + _splat(cs[L - 1])

        def place_fn(e, carry):
            w = _entry(wl, e)
            blv = w >> BL_SHIFT
            ov = plsc.load_gather(offs_b, [blv])
            plsc.store_scatter(wl2, [ov], w, mask=lane0)
            plsc.store_scatter(offs_b, [blv], ov + 1, mask=lane0)
            return carry

        lax.fori_loop(0, count, place_fn, 0)

        jvecs = [iota() + m * L for m in range(D // L)]
        spare_v = _splat(spare)

        def flush():
            pltpu.sync_copy(rowbuf, out_hbm.at[pos_ring])

        def process(start_e, end_e, buf, k0, col_off=0):
            def ebody(e, k):
                w = _entry(wl2, e)
                bv = w & ((1 << B_BITS) - 1)
                ic = ((w >> IC_SHIFT) & 255) + col_off
                vs = [plsc.load_gather(buf, [jv, ic]) for jv in jvecs]
                ss = vs[0] * vs[0]
                for v in vs[1:]:
                    ss = ss + v * v
                for msk in (8, 4, 2, 1):
                    ss = ss + _perm(ss, iota() ^ msk)
                rs = jnp.minimum(_rsqrt_vec(ss), jnp.float32(1e12))
                s = k & 63
                sv = _splat(s)
                for jv, v in zip(jvecs, vs):
                    plsc.store_scatter(rowbuf, [sv, jv], v * rs)
                plsc.store_scatter(pos_ring, [sv], bv, mask=lane0)

                @pl.when(s == 63)
                def _():
                    flush()

                return k + 1

            return lax.fori_loop(start_e, end_e, ebody, k0)

        def bucket_bounds(bl):
            st = plsc.load_gather(offs_a, [_splat(bl)])[0]
            cn = plsc.load_gather(counts, [_splat(bl)])[0]
            return st, st + cn

        def quad_fn(h, k):
            for q in range(NBUF):
                bq = NBUF * h + q

                @pl.when(bq < n_local)
                def _():
                    wait_blk(bq, bufs[q], sems[q])

                stq, enq = bucket_bounds(bq)
                enq = jnp.where(bq < n_local, enq, stq)
                k = process(stq, enq, bufs[q], k)

                @pl.when(bq + NBUF < n_local)
                def _():
                    start_blk(bq + NBUF, bufs[q], sems[q])

            return k

        k = lax.fori_loop(
            0, (n_local + NBUF - 1) // NBUF, quad_fn, jnp.int32(0))

        st_t = plsc.load_gather(offs_a, [_splat(n_local)])[0]
        k = process(st_t, count, tail_v, k, col_off=tail_off)

        rem = k & 63

        @pl.when(rem > 0)
        def _():
            rv = _splat(rem)
            for m in range(D // L):
                pv = pos_ring[pl.ds(m * L, L)]
                lanes = iota() + m * L
                pos_ring[pl.ds(m * L, L)] = jnp.where(
                    lanes < rv, pv, spare_v)
            flush()

    return body


def kernel(X, table):
    batch = X.shape[0]
    num_emb, d = table.shape
    tablet = table.T
    tail = lax.slice(table, (num_emb - OUTW, 0), (num_emb, d)).T
    out = _make_kernel(batch, num_emb)(X.astype(jnp.int32), tablet, tail)
    return out[:batch, :d]


# speedup vs baseline: 4.3242x; 1.0124x over previous
import functools

import jax
import jax.numpy as jnp
from jax import lax
from jax.experimental import pallas as pl
from jax.experimental.pallas import tpu as pltpu
from jax.experimental.pallas import tpu_sc as plsc

D = 64
L = 16
NC = 2
NS = 16
NW = NC * NS
BLKW = 128
LOG_BLKW = 7
OUTW = 128
SPARE = 8

B_BITS = 14
IC_SHIFT = B_BITS
BL_SHIFT = B_BITS + LOG_BLKW


def _rsqrt_vec(x):
    i = lax.bitcast_convert_type(x, jnp.int32)
    i = jnp.int32(0x5F3759DF) - (i >> 1)
    y = lax.bitcast_convert_type(i, jnp.float32)
    half = x * jnp.float32(0.5)
    for _ in range(3):
        y = y * (jnp.float32(1.5) - half * y * y)
    return y


def _splat(s):
    return jnp.broadcast_to(s, (L,))


def _perm(v, idx):
    return jnp.take_along_axis(v, idx, axis=0, mode="promise_in_bounds")


def _make_kernel(batch, num_emb):
    nblk = num_emb // BLKW
    tail_start = nblk * BLKW
    tail_off = tail_start - (num_emb - OUTW)
    n_vecs = batch // L
    cap = ((nblk // NW + 2 + L - 1) // L) * L
    out_rows = batch + SPARE
    mesh = plsc.VectorSubcoreMesh(
        core_axis_name="c", subcore_axis_name="s",
        num_cores=NC, num_subcores=NS,
    )

    iota = lambda: lax.iota(jnp.int32, L)

    @functools.partial(
        pl.kernel,
        out_type=jax.ShapeDtypeStruct((out_rows, OUTW), jnp.float32),
        mesh=mesh,
        scratch_types=[
            pltpu.VMEM((2048,), jnp.int32),
            pltpu.VMEM((batch,), jnp.int32),
            pltpu.VMEM((batch,), jnp.int32),
            pltpu.VMEM((cap,), jnp.int32),
            pltpu.VMEM((cap,), jnp.int32),
            pltpu.VMEM((cap,), jnp.int32),
            pltpu.VMEM((D, OUTW), jnp.float32),
            pltpu.VMEM((D, BLKW), jnp.float32),
            pltpu.VMEM((D, BLKW), jnp.float32),
            pltpu.VMEM((D, BLKW), jnp.float32),
            pltpu.VMEM((D, BLKW), jnp.float32),
            pltpu.VMEM((D, BLKW), jnp.float32),
            pltpu.VMEM((D, BLKW), jnp.float32),
            pltpu.VMEM((D, OUTW), jnp.float32),
            pltpu.VMEM((D,), jnp.int32),
            pltpu.SemaphoreType.DMA,
            pltpu.SemaphoreType.DMA,
            pltpu.SemaphoreType.DMA,
            pltpu.SemaphoreType.DMA,
            pltpu.SemaphoreType.DMA,
            pltpu.SemaphoreType.DMA,
        ],
        compiler_params=pltpu.CompilerParams(needs_layout_passes=False),
    )
    def body(x_hbm, tablet_hbm, tail_hbm, out_hbm,
             xv, wl, wl2, counts, offs_a, offs_b,
             tail_v, blk0, blk1, blk2, blk3, blk4, blk5, rowbuf, pos_ring,
             sem0, sem1, sem2, sem3, sem4, sem5):
        wid = lax.axis_index("s") * NC + lax.axis_index("c")
        c_lo = (wid * nblk) // NW
        c_hi = ((wid + 1) * nblk) // NW
        n_local = c_hi - c_lo
        is_last = wid == NW - 1
        spare = batch + (wid & (SPARE - 1))

        pltpu.sync_copy(tail_hbm, tail_v)

        def start_blk(bl, buf, sem):
            coff = pl.multiple_of((c_lo + bl) * BLKW, OUTW)
            pltpu.make_async_copy(
                tablet_hbm.at[:, pl.ds(coff, BLKW)], buf, sem
            ).start()

        def wait_blk(bl, buf, sem):
            coff = pl.multiple_of((c_lo + bl) * BLKW, OUTW)
            pltpu.make_async_copy(
                tablet_hbm.at[:, pl.ds(coff, BLKW)], buf, sem
            ).wait()

        bufs = (blk0, blk1, blk2, blk3, blk4, blk5)
        sems = (sem0, sem1, sem2, sem3, sem4, sem5)
        NBUF = 6
        for q in range(NBUF):
            start_blk(q, bufs[q], sems[q])

        lo_v = _splat(c_lo)
        hi_v = _splat(c_hi)
        last_v = jnp.broadcast_to(is_last, (L,))
        nblk_v = _splat(jnp.int32(nblk))

        XCH = 2048

        def chunk_fn(ch, off):
            pltpu.sync_copy(x_hbm.at[pl.ds(ch * XCH, XCH)], xv)

            def scan_fn(o, off):
                v = xv[pl.ds(o * L, L)]
                tc = v >> LOG_BLKW
                m = ((tc >= lo_v) & (tc < hi_v)) | (
                    last_v & (tc == nblk_v))
                word = ((iota() + (ch * XCH + o * L))
                        | ((v & (BLKW - 1)) << IC_SHIFT)
                        | ((tc - lo_v) << BL_SHIFT))
                plsc.store_compressed(
                    wl.at[pl.ds(off[0], L)], word, mask=m)
                return off + plsc.all_reduce_population_count(m)

            return lax.fori_loop(0, XCH // L, scan_fn, off)

        offv = lax.fori_loop(
            0, batch // XCH, chunk_fn, _splat(jnp.int32(0)))
        count = offv[0]

        zero_v = _splat(jnp.int32(0))
        for m in range(cap // L):
            counts[pl.ds(m * L, L)] = zero_v

        lane0 = iota() == 0

        def _entry(ref, e):
            base = (e >> 4) << 4
            v = ref[pl.ds(base, L)]
            return _perm(v, _splat(e & 15))

        def count_fn(e, carry):
            blv = _entry(wl, e) >> BL_SHIFT
            c = plsc.load_gather(counts, [blv])
            plsc.store_scatter(counts, [blv], c + 1, mask=lane0)
            return carry

        lax.fori_loop(0, count, count_fn, 0)

        carry = zero_v
        for m in range(cap // L):
            c = counts[pl.ds(m * L, L)]
            cs = plsc.cumsum(c)
            excl = cs - c + carry
            offs_a[pl.ds(m * L, L)] = excl
            offs_b[pl.ds(m * L, L)] = excl
            carry = carry + _splat(cs[L - 1])

        def place_fn(e, carry):
            w = _entry(wl, e)
            blv = w >> BL_SHIFT
            ov = plsc.load_gather(offs_b, [blv])
            plsc.store_scatter(wl2, [ov], w, mask=lane0)
            plsc.store_scatter(offs_b, [blv], ov + 1, mask=lane0)
            return carry

        lax.fori_loop(0, count, place_fn, 0)

        jvecs = [iota() + m * L for m in range(D // L)]
        spare_v = _splat(spare)

        def flush():
            pltpu.sync_copy(rowbuf, out_hbm.at[pos_ring])

        def process(start_e, end_e, buf, k0, col_off=0):
            def ebody(e, k):
                w = _entry(wl2, e)
                bv = w & ((1 << B_BITS) - 1)
                ic = ((w >> IC_SHIFT) & (BLKW - 1)) + col_off
                vs = [plsc.load_gather(buf, [jv, ic]) for jv in jvecs]
                ss = vs[0] * vs[0]
                for v in vs[1:]:
                    ss = ss + v * v
                for msk in (8, 4, 2, 1):
                    ss = ss + _perm(ss, iota() ^ msk)
                rs = jnp.minimum(_rsqrt_vec(ss), jnp.float32(1e12))
                s = k & 63
                sv = _splat(s)
                for jv, v in zip(jvecs, vs):
                    plsc.store_scatter(rowbuf, [sv, jv], v * rs)
                plsc.store_scatter(pos_ring, [sv], bv, mask=lane0)

                @pl.when(s == 63)
                def _():
                    flush()

                return k + 1

            return lax.fori_loop(start_e, end_e, ebody, k0)

        def bucket_bounds(bl):
            st = plsc.load_gather(offs_a, [_splat(bl)])[0]
            cn = plsc.load_gather(counts, [_splat(bl)])[0]
            return st, st + cn

        def quad_fn(h, k):
            for q in range(NBUF):
                bq = NBUF * h + q

                @pl.when(bq < n_local)
                def _():
                    wait_blk(bq, bufs[q], sems[q])

                stq, enq = bucket_bounds(bq)
                enq = jnp.where(bq < n_local, enq, stq)
                k = process(stq, enq, bufs[q], k)

                @pl.when(bq + NBUF < n_local)
                def _():
                    start_blk(bq + NBUF, bufs[q], sems[q])

            return k

        k = lax.fori_loop(
            0, (n_local + NBUF - 1) // NBUF, quad_fn, jnp.int32(0))

        st_t = plsc.load_gather(offs_a, [_splat(n_local)])[0]
        k = process(st_t, count, tail_v, k, col_off=tail_off)

        rem = k & 63

        @pl.when(rem > 0)
        def _():
            rv = _splat(rem)
            for m in range(D // L):
                pv = pos_ring[pl.ds(m * L, L)]
                lanes = iota() + m * L
                pos_ring[pl.ds(m * L, L)] = jnp.where(
                    lanes < rv, pv, spare_v)
            flush()

    return body


def kernel(X, table):
    batch = X.shape[0]
    num_emb, d = table.shape
    tablet = table.T
    tail = lax.slice(table, (num_emb - OUTW, 0), (num_emb, d)).T
    out = _make_kernel(batch, num_emb)(X.astype(jnp.int32), tablet, tail)
    return out[:batch, :d]


# speedup vs baseline: 4.6471x; 1.0747x over previous
import functools

import jax
import jax.numpy as jnp
from jax import lax
from jax.experimental import pallas as pl
from jax.experimental.pallas import tpu as pltpu
from jax.experimental.pallas import tpu_sc as plsc

D = 64
L = 16
NC = 2
NS = 16
NW = NC * NS
BLKW = 128
LOG_BLKW = 7
OUTW = 128
SPARE = 8

B_BITS = 14
IC_SHIFT = B_BITS
BL_SHIFT = B_BITS + LOG_BLKW


def _rsqrt_vec(x):
    i = lax.bitcast_convert_type(x, jnp.int32)
    i = jnp.int32(0x5F3759DF) - (i >> 1)
    y = lax.bitcast_convert_type(i, jnp.float32)
    half = x * jnp.float32(0.5)
    for _ in range(3):
        y = y * (jnp.float32(1.5) - half * y * y)
    return y


def _splat(s):
    return jnp.broadcast_to(s, (L,))


def _perm(v, idx):
    return jnp.take_along_axis(v, idx, axis=0, mode="promise_in_bounds")


def _make_kernel(batch, num_emb):
    nblk = num_emb // BLKW
    tail_start = nblk * BLKW
    tail_off = tail_start - (num_emb - OUTW)
    n_vecs = batch // L
    cap = ((nblk // NW + 2 + L - 1) // L) * L + L
    out_rows = batch + SPARE
    mesh = plsc.VectorSubcoreMesh(
        core_axis_name="c", subcore_axis_name="s",
        num_cores=NC, num_subcores=NS,
    )

    iota = lambda: lax.iota(jnp.int32, L)

    @functools.partial(
        pl.kernel,
        out_type=jax.ShapeDtypeStruct((out_rows, OUTW), jnp.float32),
        mesh=mesh,
        scratch_types=[
            pltpu.VMEM((2048,), jnp.int32),
            pltpu.VMEM((batch,), jnp.int32),
            pltpu.VMEM((batch,), jnp.int32),
            pltpu.VMEM((cap,), jnp.int32),
            pltpu.VMEM((cap,), jnp.int32),
            pltpu.VMEM((cap,), jnp.int32),
            pltpu.VMEM((cap,), jnp.int32),
            pltpu.VMEM((D, OUTW), jnp.float32),
            pltpu.VMEM((D, BLKW), jnp.float32),
            pltpu.VMEM((D, BLKW), jnp.float32),
            pltpu.VMEM((D, BLKW), jnp.float32),
            pltpu.VMEM((D, BLKW), jnp.float32),
            pltpu.VMEM((D, BLKW), jnp.float32),
            pltpu.VMEM((D, BLKW), jnp.float32),
            pltpu.VMEM((D, OUTW), jnp.float32),
            pltpu.VMEM((D,), jnp.int32),
            pltpu.SemaphoreType.DMA,
            pltpu.SemaphoreType.DMA,
            pltpu.SemaphoreType.DMA,
            pltpu.SemaphoreType.DMA,
            pltpu.SemaphoreType.DMA,
            pltpu.SemaphoreType.DMA,
        ],
        compiler_params=pltpu.CompilerParams(needs_layout_passes=False),
    )
    def body(x_hbm, tablet_hbm, tail_hbm, out_hbm,
             xv, wl, wl2, counts, offs_a, offs_b, occ,
             tail_v, blk0, blk1, blk2, blk3, blk4, blk5, rowbuf, pos_ring,
             sem0, sem1, sem2, sem3, sem4, sem5):
        wid = lax.axis_index("s") * NC + lax.axis_index("c")
        c_lo = (wid * nblk) // NW
        c_hi = ((wid + 1) * nblk) // NW
        n_local = c_hi - c_lo
        is_last = wid == NW - 1
        spare = batch + (wid & (SPARE - 1))

        pltpu.sync_copy(tail_hbm, tail_v)

        def start_blk(bl, buf, sem):
            coff = pl.multiple_of((c_lo + bl) * BLKW, OUTW)
            pltpu.make_async_copy(
                tablet_hbm.at[:, pl.ds(coff, BLKW)], buf, sem
            ).start()

        def wait_blk(bl, buf, sem):
            coff = pl.multiple_of((c_lo + bl) * BLKW, OUTW)
            pltpu.make_async_copy(
                tablet_hbm.at[:, pl.ds(coff, BLKW)], buf, sem
            ).wait()

        bufs = (blk0, blk1, blk2, blk3, blk4, blk5)
        sems = (sem0, sem1, sem2, sem3, sem4, sem5)
        NBUF = 6

        def occ_at(p):
            v = plsc.load_gather(occ, [_splat(p)])
            return jnp.where(_splat(p < n_occ), v, _splat(jnp.int32(0)))[0]

        lo_v = _splat(c_lo)
        hi_v = _splat(c_hi)
        last_v = jnp.broadcast_to(is_last, (L,))
        nblk_v = _splat(jnp.int32(nblk))

        XCH = 2048

        def chunk_fn(ch, off):
            pltpu.sync_copy(x_hbm.at[pl.ds(ch * XCH, XCH)], xv)

            def scan_fn(o, off):
                v = xv[pl.ds(o * L, L)]
                tc = v >> LOG_BLKW
                m = ((tc >= lo_v) & (tc < hi_v)) | (
                    last_v & (tc == nblk_v))
                word = ((iota() + (ch * XCH + o * L))
                        | ((v & (BLKW - 1)) << IC_SHIFT)
                        | ((tc - lo_v) << BL_SHIFT))
                plsc.store_compressed(
                    wl.at[pl.ds(off[0], L)], word, mask=m)
                return off + plsc.all_reduce_population_count(m)

            return lax.fori_loop(0, XCH // L, scan_fn, off)

        offv = lax.fori_loop(
            0, batch // XCH, chunk_fn, _splat(jnp.int32(0)))
        count = offv[0]

        zero_v = _splat(jnp.int32(0))
        for m in range(cap // L):
            counts[pl.ds(m * L, L)] = zero_v

        lane0 = iota() == 0

        def _entry(ref, e):
            base = (e >> 4) << 4
            v = ref[pl.ds(base, L)]
            return _perm(v, _splat(e & 15))

        def count_fn(e, carry):
            blv = _entry(wl, e) >> BL_SHIFT
            c = plsc.load_gather(counts, [blv])
            plsc.store_scatter(counts, [blv], c + 1, mask=lane0)
            return carry

        lax.fori_loop(0, count, count_fn, 0)

        carry = zero_v
        for m in range(cap // L):
            c = counts[pl.ds(m * L, L)]
            cs = plsc.cumsum(c)
            excl = cs - c + carry
            offs_a[pl.ds(m * L, L)] = excl
            offs_b[pl.ds(m * L, L)] = excl
            carry = carry + _splat(cs[L - 1])

        n_local_v = _splat(n_local)

        def occ_fn(m, noff):
            c = counts[pl.ds(m * L, L)]
            lanes = iota() + m * L
            mk = (c > 0) & (lanes < n_local_v)
            plsc.store_compressed(occ.at[pl.ds(noff[0], L)], lanes, mask=mk)
            return noff + plsc.all_reduce_population_count(mk)

        noffv = lax.fori_loop(0, cap // L, occ_fn, _splat(jnp.int32(0)))
        n_occ = noffv[0]

        def place_fn(e, carry):
            w = _entry(wl, e)
            blv = w >> BL_SHIFT
            ov = plsc.load_gather(offs_b, [blv])
            plsc.store_scatter(wl2, [ov], w, mask=lane0)
            plsc.store_scatter(offs_b, [blv], ov + 1, mask=lane0)
            return carry

        lax.fori_loop(0, count, place_fn, 0)

        jvecs = [iota() + m * L for m in range(D // L)]
        spare_v = _splat(spare)

        def flush():
            pltpu.sync_copy(rowbuf, out_hbm.at[pos_ring])

        def process(start_e, end_e, buf, k0, col_off=0):
            def ebody(e, k):
                w = _entry(wl2, e)
                bv = w & ((1 << B_BITS) - 1)
                ic = ((w >> IC_SHIFT) & (BLKW - 1)) + col_off
                vs = [plsc.load_gather(buf, [jv, ic]) for jv in jvecs]
                ss = vs[0] * vs[0]
                for v in vs[1:]:
                    ss = ss + v * v
                for msk in (8, 4, 2, 1):
                    ss = ss + _perm(ss, iota() ^ msk)
                rs = jnp.minimum(_rsqrt_vec(ss), jnp.float32(1e12))
                s = k & 63
                sv = _splat(s)
                for jv, v in zip(jvecs, vs):
                    plsc.store_scatter(rowbuf, [sv, jv], v * rs)
                plsc.store_scatter(pos_ring, [sv], bv, mask=lane0)

                @pl.when(s == 63)
                def _():
                    flush()

                return k + 1

            return lax.fori_loop(start_e, end_e, ebody, k0)

        def bucket_bounds(bl):
            st = plsc.load_gather(offs_a, [_splat(bl)])[0]
            cn = plsc.load_gather(counts, [_splat(bl)])[0]
            return st, st + cn

        for q in range(NBUF):
            @pl.when(q < n_occ)
            def _():
                start_blk(occ_at(q), bufs[q], sems[q])

        def ring_fn(h, k):
            for q in range(NBUF):
                bq = NBUF * h + q
                cq = occ_at(bq)

                @pl.when(bq < n_occ)
                def _():
                    wait_blk(cq, bufs[q], sems[q])

                stq, enq = bucket_bounds(cq)
                enq = jnp.where(bq < n_occ, enq, stq)
                k = process(stq, enq, bufs[q], k)

                @pl.when(bq + NBUF < n_occ)
                def _():
                    start_blk(occ_at(bq + NBUF), bufs[q], sems[q])

            return k

        k = lax.fori_loop(
            0, (n_occ + NBUF - 1) // NBUF, ring_fn, jnp.int32(0))

        st_t = plsc.load_gather(offs_a, [_splat(n_local)])[0]
        k = process(st_t, count, tail_v, k, col_off=tail_off)

        rem = k & 63

        @pl.when(rem > 0)
        def _():
            rv = _splat(rem)
            for m in range(D // L):
                pv = pos_ring[pl.ds(m * L, L)]
                lanes = iota() + m * L
                pos_ring[pl.ds(m * L, L)] = jnp.where(
                    lanes < rv, pv, spare_v)
            flush()

    return body


def kernel(X, table):
    batch = X.shape[0]
    num_emb, d = table.shape
    tablet = table.T
    tail = lax.slice(table, (num_emb - OUTW, 0), (num_emb, d)).T
    out = _make_kernel(batch, num_emb)(X.astype(jnp.int32), tablet, tail)
    return out[:batch, :d]
